# baseline (speedup 1.0000x reference)
import jax
import jax.numpy as jnp
from jax.experimental import pallas as pl

_Q = 1024
_K = 65536
_TOPK = 500
_TILE = 2048


def _neg_dist_kernel(hand_ref, bT_ref, a2_ref, b2_ref, out_ref):
    h = hand_ref[:, :]
    bt = bT_ref[:, :]
    ab = jnp.dot(h.astype(jnp.bfloat16), bt.astype(jnp.bfloat16),
                 preferred_element_type=jnp.float32)
    d2 = (a2_ref[:, :] + b2_ref[:, :]) - 2.0 * ab
    out_ref[:, :] = -jnp.sqrt(jnp.maximum(d2, 1e-12))


def _gather_weight_kernel(qi_ref, ji_ref, hand_ref, body_ref, w_ref, out_ref):
    hp = jax.lax.Precision.HIGHEST
    iota_q = jax.lax.broadcasted_iota(jnp.int32, (1, _Q), 1)
    oh_h = (qi_ref[:, :] == iota_q).astype(jnp.float32)
    hacc = jnp.dot(oh_h, hand_ref[:, :], precision=hp,
                   preferred_element_type=jnp.float32)

    def body_fn(t, acc):
        start = t * _TILE
        iota_j = jax.lax.broadcasted_iota(jnp.int32, (1, _TILE), 1) + start
        oh_b = (ji_ref[:, :] == iota_j).astype(jnp.float32)
        chunk = body_ref[pl.ds(start, _TILE), :]
        return acc + jnp.dot(oh_b, chunk, precision=hp,
                             preferred_element_type=jnp.float32)

    bacc = jax.lax.fori_loop(0, _K // _TILE, body_fn,
                             jnp.zeros((_TOPK, 3), jnp.float32))
    diff = jnp.abs(hacc - bacc)
    out_ref[:, :] = jnp.sum(diff * w_ref[0:1, :], axis=1, keepdims=True)


def kernel(hand_verts, body_verts, region_weights):
    bT = body_verts.T
    a2 = jnp.sum(hand_verts * hand_verts, axis=-1)[:, None]
    b2 = jnp.sum(body_verts * body_verts, axis=-1)[None, :]

    neg_d = pl.pallas_call(
        _neg_dist_kernel,
        grid=(_K // _TILE,),
        in_specs=[
            pl.BlockSpec((_Q, 3), lambda i: (0, 0)),
            pl.BlockSpec((3, _TILE), lambda i: (0, i)),
            pl.BlockSpec((_Q, 1), lambda i: (0, 0)),
            pl.BlockSpec((1, _TILE), lambda i: (0, i)),
        ],
        out_specs=pl.BlockSpec((_Q, _TILE), lambda i: (0, i)),
        out_shape=jax.ShapeDtypeStruct((_Q, _K), jnp.float32),
    )(hand_verts, bT, a2, b2)

    _, top_idx = jax.lax.top_k(neg_d.reshape(-1), _TOPK)
    q_idx = (top_idx // _K).astype(jnp.int32).reshape(_TOPK, 1)
    j_idx = (top_idx % _K).astype(jnp.int32).reshape(_TOPK, 1)

    out = pl.pallas_call(
        _gather_weight_kernel,
        out_shape=jax.ShapeDtypeStruct((_TOPK, 1), jnp.float32),
    )(q_idx, j_idx, hand_verts, body_verts, region_weights.reshape(1, 3))

    return out.reshape(_TOPK)


# speedup vs baseline: 2.3616x; 2.3616x over previous
import jax
import jax.numpy as jnp
from jax.experimental import pallas as pl

_Q = 1024
_K = 65536
_TOPK = 500
_TILE = 2048


def _neg_dist_kernel(hand_ref, bT_ref, a2_ref, b2_ref, out_ref):
    h = hand_ref[:, :]
    bt = bT_ref[:, :]
    ab = jnp.dot(h.astype(jnp.bfloat16), bt.astype(jnp.bfloat16),
                 preferred_element_type=jnp.float32)
    d2 = (a2_ref[:, :] + b2_ref[:, :]) - 2.0 * ab
    out_ref[:, :] = -jnp.sqrt(jnp.maximum(d2, 1e-12))


def _gather_weight_kernel(qi_ref, ji_ref, hand_ref, body_ref, w_ref, out_ref):
    hp = jax.lax.Precision.HIGHEST
    iota_q = jax.lax.broadcasted_iota(jnp.int32, (1, _Q), 1)
    oh_h = (qi_ref[:, :] == iota_q).astype(jnp.float32)
    hacc = jnp.dot(oh_h, hand_ref[:, :], precision=hp,
                   preferred_element_type=jnp.float32)

    def body_fn(t, acc):
        start = t * _TILE
        iota_j = jax.lax.broadcasted_iota(jnp.int32, (1, _TILE), 1) + start
        oh_b = (ji_ref[:, :] == iota_j).astype(jnp.float32)
        chunk = body_ref[pl.ds(start, _TILE), :]
        return acc + jnp.dot(oh_b, chunk, precision=hp,
                             preferred_element_type=jnp.float32)

    bacc = jax.lax.fori_loop(0, _K // _TILE, body_fn,
                             jnp.zeros((_TOPK, 3), jnp.float32))
    diff = jnp.abs(hacc - bacc)
    out_ref[:, :] = jnp.sum(diff * w_ref[0:1, :], axis=1, keepdims=True)


def kernel(hand_verts, body_verts, region_weights):
    bT = body_verts.T
    a2 = jnp.sum(hand_verts * hand_verts, axis=-1)[:, None]
    b2 = jnp.sum(body_verts * body_verts, axis=-1)[None, :]

    neg_d = pl.pallas_call(
        _neg_dist_kernel,
        grid=(_K // _TILE,),
        in_specs=[
            pl.BlockSpec((_Q, 3), lambda i: (0, 0)),
            pl.BlockSpec((3, _TILE), lambda i: (0, i)),
            pl.BlockSpec((_Q, 1), lambda i: (0, 0)),
            pl.BlockSpec((1, _TILE), lambda i: (0, i)),
        ],
        out_specs=pl.BlockSpec((_Q, _TILE), lambda i: (0, i)),
        out_shape=jax.ShapeDtypeStruct((_Q, _K), jnp.float32),
    )(hand_verts, bT, a2, b2)

    row_vals, row_idx = jax.lax.top_k(neg_d, _TOPK)
    flat_cand_idx = (jnp.arange(_Q, dtype=jnp.int32)[:, None] * _K
                     + row_idx).reshape(-1)
    _, cand_pos = jax.lax.top_k(row_vals.reshape(-1), _TOPK)
    top_idx = flat_cand_idx[cand_pos]
    q_idx = (top_idx // _K).astype(jnp.int32).reshape(_TOPK, 1)
    j_idx = (top_idx % _K).astype(jnp.int32).reshape(_TOPK, 1)

    out = pl.pallas_call(
        _gather_weight_kernel,
        out_shape=jax.ShapeDtypeStruct((_TOPK, 1), jnp.float32),
    )(q_idx, j_idx, hand_verts, body_verts, region_weights.reshape(1, 3))

    return out.reshape(_TOPK)
